# baseline (speedup 1.0000x reference)
import jax
import jax.numpy as jnp
from jax.experimental import pallas as pl
from jax.experimental.pallas import tpu as pltpu
from jax.experimental.pallas import tpu_sc as plsc

_B = 4096
_V = 100000
_NTOP = 1024
_BT = 256
_VT = 4096
_NB = _B // _BT
_NV = -(-_V // _VT)
_GATHER_W = 128


_LANES = 128
_NROWS = _B * _V // _LANES


def _sc_gather(x_rows, row_idx):
    mesh = plsc.VectorSubcoreMesh(core_axis_name="core", subcore_axis_name="subcore")

    @pl.kernel(out_type=jax.ShapeDtypeStruct((_B, _LANES), jnp.float32),
               mesh=mesh)
    def gather_kernel(x_hbm, i_hbm, o_hbm):
        def body(i_vmem, o_vmem):
            pltpu.sync_copy(x_hbm.at[i_vmem.at[0]], o_vmem)

        pltpu.emit_pipeline(
            body,
            grid=(_B // _GATHER_W,),
            in_specs=[pl.BlockSpec((1, _GATHER_W), lambda i: (0, i))],
            out_specs=[pl.BlockSpec((_GATHER_W, _LANES), lambda i: (i, 0))],
            core_axis_name="subcore",
            dimension_semantics=(pltpu.PARALLEL,),
        )(i_hbm, o_hbm)

    return gather_kernel(x_rows, row_idx.reshape(1, _B))


def _stream_kernel(params_ref, tokens_ref, ids_ref, xt_rows_ref, lane_ref,
                   x_ref, out_ref, m_ref, s_ref):
    j = pl.program_id(1)

    @pl.when(j == 0)
    def _():
        m_ref[...] = jnp.full((_BT, 1), -jnp.inf, jnp.float32)
        s_ref[...] = jnp.zeros((_BT, 1), jnp.float32)

    tile = x_ref[...]
    col = j * _VT + jax.lax.broadcasted_iota(jnp.int32, (_BT, _VT), 1)
    tile = jnp.where(col < _V, tile, -jnp.inf)

    m_old = m_ref[...]
    m_new = jnp.maximum(m_old, jnp.max(tile, axis=1, keepdims=True))
    s_ref[...] = (s_ref[...] * jnp.exp(m_old - m_new)
                  + jnp.sum(jnp.exp(tile - m_new), axis=1, keepdims=True))
    m_ref[...] = m_new

    @pl.when(j == _NV - 1)
    def _():
        toks = tokens_ref[...]
        ids = ids_ref[...]
        mask = jnp.any(toks[:, None] == ids[None, :], axis=1)
        lane = jax.lax.broadcasted_iota(jnp.int32, (_BT, _LANES), 1)
        xt = jnp.sum(jnp.where(lane == lane_ref[...][:, None],
                               xt_rows_ref[...], 0.0), axis=1)
        g = jnp.exp(xt - m_ref[:, 0]) / s_ref[:, 0]
        w = jnp.where(mask, params_ref[2], params_ref[0])
        b = jnp.where(mask, params_ref[3], params_ref[1])
        out_ref[...] = jax.nn.sigmoid(g * w + b)


def kernel(x, tokens, top_token_ids, gen_w, gen_b, top_w, top_b):
    tokens = tokens.astype(jnp.int32)
    ids = top_token_ids.astype(jnp.int32)
    flat_idx = tokens + jnp.arange(_B, dtype=jnp.int32) * _V
    row_idx = flat_idx // _LANES
    lane_idx = flat_idx % _LANES
    xt_rows = _sc_gather(x.reshape(_NROWS, _LANES), row_idx)
    params = jnp.concatenate([gen_w.reshape(-1), gen_b.reshape(-1),
                              top_w.reshape(-1), top_b.reshape(-1)])
    return pl.pallas_call(
        _stream_kernel,
        grid=(_NB, _NV),
        in_specs=[
            pl.BlockSpec(memory_space=pltpu.SMEM),
            pl.BlockSpec((_BT,), lambda i, j: (i,)),
            pl.BlockSpec((_NTOP,), lambda i, j: (0,)),
            pl.BlockSpec((_BT, _LANES), lambda i, j: (i, 0)),
            pl.BlockSpec((_BT,), lambda i, j: (i,)),
            pl.BlockSpec((_BT, _VT), lambda i, j: (i, j)),
        ],
        out_specs=pl.BlockSpec((_BT,), lambda i, j: (i,)),
        out_shape=jax.ShapeDtypeStruct((_B,), jnp.float32),
        scratch_shapes=[pltpu.VMEM((_BT, 1), jnp.float32),
                        pltpu.VMEM((_BT, 1), jnp.float32)],
        compiler_params=pltpu.CompilerParams(
            dimension_semantics=("parallel", "arbitrary")),
    )(params, tokens, ids, xt_rows, lane_idx, x)


# speedup vs baseline: 2.0268x; 2.0268x over previous
import jax
import jax.numpy as jnp
from jax.experimental import pallas as pl
from jax.experimental.pallas import tpu as pltpu

_B = 4096
_V = 100000
_NTOP = 1024
_BT = 256
_VT = 4096
_NB = _B // _BT
_NV = -(-_V // _VT)


def _stream_kernel(params_ref, tokens_ref, ids_ref, x_ref, out_ref,
                   m_ref, s_ref, xt_ref):
    j = pl.program_id(1)

    @pl.when(j == 0)
    def _():
        m_ref[...] = jnp.full((_BT, 1), -jnp.inf, jnp.float32)
        s_ref[...] = jnp.zeros((_BT, 1), jnp.float32)
        xt_ref[...] = jnp.zeros((_BT, 1), jnp.float32)

    tile = x_ref[...]
    loc = jax.lax.broadcasted_iota(jnp.int32, (_BT, _VT), 1)

    tloc = tokens_ref[...][:, None] - j * _VT
    xt_ref[...] += jnp.sum(jnp.where(loc == tloc, tile, 0.0),
                           axis=1, keepdims=True)

    tile = jnp.where(loc < _V - j * _VT, tile, -jnp.inf)

    m_old = m_ref[...]
    m_new = jnp.maximum(m_old, jnp.max(tile, axis=1, keepdims=True))
    s_ref[...] = (s_ref[...] * jnp.exp(m_old - m_new)
                  + jnp.sum(jnp.exp(tile - m_new), axis=1, keepdims=True))
    m_ref[...] = m_new

    @pl.when(j == _NV - 1)
    def _():
        toks = tokens_ref[...]
        ids = ids_ref[...]
        mask = jnp.any(toks[:, None] == ids[None, :], axis=1)
        g = jnp.exp(xt_ref[:, 0] - m_ref[:, 0]) / s_ref[:, 0]
        w = jnp.where(mask, params_ref[2], params_ref[0])
        b = jnp.where(mask, params_ref[3], params_ref[1])
        out_ref[...] = jax.nn.sigmoid(g * w + b)


def kernel(x, tokens, top_token_ids, gen_w, gen_b, top_w, top_b):
    tokens = tokens.astype(jnp.int32)
    ids = top_token_ids.astype(jnp.int32)
    params = jnp.concatenate([gen_w.reshape(-1), gen_b.reshape(-1),
                              top_w.reshape(-1), top_b.reshape(-1)])
    return pl.pallas_call(
        _stream_kernel,
        grid=(_NB, _NV),
        in_specs=[
            pl.BlockSpec(memory_space=pltpu.SMEM),
            pl.BlockSpec((_BT,), lambda i, j: (i,)),
            pl.BlockSpec((_NTOP,), lambda i, j: (0,)),
            pl.BlockSpec((_BT, _VT), lambda i, j: (i, j)),
        ],
        out_specs=pl.BlockSpec((_BT,), lambda i, j: (i,)),
        out_shape=jax.ShapeDtypeStruct((_B,), jnp.float32),
        scratch_shapes=[pltpu.VMEM((_BT, 1), jnp.float32),
                        pltpu.VMEM((_BT, 1), jnp.float32),
                        pltpu.VMEM((_BT, 1), jnp.float32)],
        compiler_params=pltpu.CompilerParams(
            dimension_semantics=("parallel", "arbitrary")),
    )(params, tokens, ids, x)
